# baseline (speedup 1.0000x reference)
import jax
import jax.numpy as jnp
from jax import lax
from jax.experimental import pallas as pl
from jax.experimental.pallas import tpu as pltpu
from jax.experimental.pallas import tpu_sc as plsc

_N = 10000
_E = 320000
_NC = 2
_NS = 16
_L = 16
_TILE = 128
_EPC = _E // _NC
_TPW = (_EPC // _TILE) // _NS
_COLS = _TPW * _TILE
_COLS_LAST = _EPC - 15 * _COLS
_UNROLL = 8
_NCHUNK = 2


def _scan_chunks(edge_hbm, b0, b1, s0, s1, base, ccols, acc):
    bufs, sems = (b0, b1), (s0, s1)
    pltpu.async_copy(edge_hbm.at[:, pl.ds(base, ccols)],
                     b0.at[:, pl.ds(0, ccols)], s0)
    for k in range(_NCHUNK):
        nxt = k + 1
        if nxt < _NCHUNK:
            pltpu.async_copy(
                edge_hbm.at[:, pl.ds(base + nxt * ccols, ccols)],
                bufs[nxt % 2].at[:, pl.ds(0, ccols)], sems[nxt % 2])
        pltpu.make_async_copy(
            edge_hbm.at[:, pl.ds(base + k * ccols, ccols)],
            bufs[k % 2].at[:, pl.ds(0, ccols)], sems[k % 2]).wait()
        buf = bufs[k % 2]

        def step(i, a):
            off = i * (_L * _UNROLL)
            for u in range(_UNROLL):
                v = buf[1, pl.ds(off + u * _L, _L)]
                m = plsc.bitcast(v, jnp.uint32) < jnp.uint32(_N)
                a = a + plsc.all_reduce_population_count(m)
            return a

        acc = lax.fori_loop(0, ccols // (_L * _UNROLL), step, acc)
    return acc


def _sc_body(edge_hbm, alpha_hbm, out_hbm, buf0_v, buf1_v, alpha_v, part_v,
             fin_v, mat_v, acc_sh, sem0, sem1, sem_a):
    cid = lax.axis_index("c")
    wid = lax.axis_index("s")
    cbase = cid * _EPC
    zero = jnp.zeros((_L,), jnp.int32)

    @pl.when((wid == 0) & (cid == 0))
    def _prefetch_alpha():
        pltpu.async_copy(alpha_hbm, alpha_v.at[pl.ds(0, 1)], sem_a)

    @pl.when(wid < 15)
    def _main():
        part_v[...] = _scan_chunks(edge_hbm, buf0_v, buf1_v, sem0, sem1,
                                   cbase + wid * _COLS, _COLS // _NCHUNK,
                                   zero)

    @pl.when(wid == 15)
    def _tail():
        part_v[...] = _scan_chunks(edge_hbm, buf0_v, buf1_v, sem0, sem1,
                                   cbase + 15 * _COLS,
                                   _COLS_LAST // _NCHUNK, zero)

    pltpu.sync_copy(part_v, acc_sh.at[pl.ds(wid * _L, _L)])
    plsc.subcore_barrier()

    @pl.when(wid == 0)
    def _finalize():
        pltpu.sync_copy(acc_sh, mat_v)

        tot_vec = mat_v[pl.ds(0, _L)]
        for w in range(1, _NS):
            tot_vec = tot_vec + mat_v[pl.ds(w * _L, _L)]
        lane = lax.iota(jnp.int32, _L)
        tr1 = tot_vec.astype(jnp.float32) * (0.5 / _N)

        @pl.when(cid == 0)
        def _with_gate():
            pltpu.make_async_copy(alpha_hbm, alpha_v.at[pl.ds(0, 1)],
                                  sem_a).wait()
            av = alpha_v[...]
            gate = 1.0 / (1.0 + jnp.exp(-av))
            fin_v[...] = jnp.where(lane == 9, tr1,
                                   jnp.where(lane == 0, gate, 0.0))
            pltpu.sync_copy(fin_v, out_hbm.at[pl.ds(0, _L)])

        @pl.when(cid == 1)
        def _count_only():
            fin_v[...] = jnp.where(lane == 9, tr1, 0.0)
            pltpu.sync_copy(fin_v, out_hbm.at[pl.ds(_L, _L)])


_sc_call = pl.kernel(
    _sc_body,
    out_type=jax.ShapeDtypeStruct((_NC * _L,), jnp.float32),
    mesh=plsc.VectorSubcoreMesh(
        core_axis_name="c", subcore_axis_name="s", num_cores=_NC),
    compiler_params=pltpu.CompilerParams(
        needs_layout_passes=False, skip_device_barrier=True),
    scratch_types=[
        pltpu.VMEM((2, _COLS_LAST // _NCHUNK), jnp.int32),
        pltpu.VMEM((2, _COLS_LAST // _NCHUNK), jnp.int32),
        pltpu.VMEM((_L,), jnp.float32),
        pltpu.VMEM((_L,), jnp.int32),
        pltpu.VMEM((_L,), jnp.float32),
        pltpu.VMEM((_NS * _L,), jnp.int32),
        pltpu.VMEM_SHARED((_NS * _L,), jnp.int32),
        pltpu.SemaphoreType.DMA,
        pltpu.SemaphoreType.DMA,
        pltpu.SemaphoreType.DMA,
    ],
)


def kernel(x, edge_index, W1, b1, W2, b2, alpha):
    out32 = _sc_call(edge_index, alpha)
    tr = out32[8:10] + out32[24:26]
    gates = out32[0:1].reshape(1, 1)
    return (tr, gates)


# speedup vs baseline: 1.0249x; 1.0249x over previous
import jax
import jax.numpy as jnp
from jax import lax
from jax.experimental import pallas as pl
from jax.experimental.pallas import tpu as pltpu
from jax.experimental.pallas import tpu_sc as plsc

_N = 10000
_E = 320000
_NS = 16
_L = 16
_TILE = 128
_TPW = (_E // _TILE) // _NS
_COLS = _TPW * _TILE
_COLS_LAST = _E - 15 * _COLS
_UNROLL = 8
_NCHUNK = 2


def _scan_chunks(edge_hbm, b0, b1, s0, s1, base, ccols, acc):
    bufs, sems = (b0, b1), (s0, s1)
    pltpu.async_copy(edge_hbm.at[:, pl.ds(base, ccols)],
                     b0.at[:, pl.ds(0, ccols)], s0)
    for k in range(_NCHUNK):
        nxt = k + 1
        if nxt < _NCHUNK:
            pltpu.async_copy(
                edge_hbm.at[:, pl.ds(base + nxt * ccols, ccols)],
                bufs[nxt % 2].at[:, pl.ds(0, ccols)], sems[nxt % 2])
        pltpu.make_async_copy(
            edge_hbm.at[:, pl.ds(base + k * ccols, ccols)],
            bufs[k % 2].at[:, pl.ds(0, ccols)], sems[k % 2]).wait()
        buf = bufs[k % 2]

        def step(i, a):
            off = i * (_L * _UNROLL)
            for u in range(_UNROLL):
                v = buf[1, pl.ds(off + u * _L, _L)]
                m = plsc.bitcast(v, jnp.uint32) < jnp.uint32(_N)
                a = a + plsc.all_reduce_population_count(m)
            return a

        acc = lax.fori_loop(0, ccols // (_L * _UNROLL), step, acc)
    return acc


def _sc_body(edge_hbm, alpha_hbm, out_hbm, buf0_v, buf1_v, alpha_v, part_v,
             fin_v, mat_v, acc_sh, sem0, sem1, sem_a):
    wid = lax.axis_index("s")
    zero = jnp.zeros((_L,), jnp.int32)

    @pl.when(wid == 0)
    def _prefetch_alpha():
        pltpu.async_copy(alpha_hbm, alpha_v.at[pl.ds(0, 1)], sem_a)

    @pl.when(wid < 15)
    def _main():
        part_v[...] = _scan_chunks(edge_hbm, buf0_v, buf1_v, sem0, sem1,
                                   wid * _COLS, _COLS // _NCHUNK, zero)

    @pl.when(wid == 15)
    def _tail():
        part_v[...] = _scan_chunks(edge_hbm, buf0_v, buf1_v, sem0, sem1,
                                   15 * _COLS, _COLS_LAST // _NCHUNK, zero)

    pltpu.sync_copy(part_v, acc_sh.at[pl.ds(wid * _L, _L)])
    plsc.subcore_barrier()

    @pl.when(wid == 0)
    def _finalize():
        pltpu.sync_copy(acc_sh, mat_v)

        tot_vec = mat_v[pl.ds(0, _L)]
        for w in range(1, _NS):
            tot_vec = tot_vec + mat_v[pl.ds(w * _L, _L)]
        pltpu.make_async_copy(alpha_hbm, alpha_v.at[pl.ds(0, 1)],
                              sem_a).wait()
        av = alpha_v[...]
        gate = 1.0 / (1.0 + jnp.exp(-av))
        lane = lax.iota(jnp.int32, _L)
        tr1 = tot_vec.astype(jnp.float32) * (0.5 / _N)
        fin_v[...] = jnp.where(lane == 9, tr1,
                               jnp.where(lane == 0, gate, 0.0))
        pltpu.sync_copy(fin_v, out_hbm)


_sc_call = pl.kernel(
    _sc_body,
    out_type=jax.ShapeDtypeStruct((_L,), jnp.float32),
    mesh=plsc.VectorSubcoreMesh(
        core_axis_name="c", subcore_axis_name="s", num_cores=1),
    compiler_params=pltpu.CompilerParams(
        needs_layout_passes=False, skip_device_barrier=True),
    scratch_types=[
        pltpu.VMEM((2, _COLS_LAST // _NCHUNK), jnp.int32),
        pltpu.VMEM((2, _COLS_LAST // _NCHUNK), jnp.int32),
        pltpu.VMEM((_L,), jnp.float32),
        pltpu.VMEM((_L,), jnp.int32),
        pltpu.VMEM((_L,), jnp.float32),
        pltpu.VMEM((_NS * _L,), jnp.int32),
        pltpu.VMEM_SHARED((_NS * _L,), jnp.int32),
        pltpu.SemaphoreType.DMA,
        pltpu.SemaphoreType.DMA,
        pltpu.SemaphoreType.DMA,
    ],
)


def kernel(x, edge_index, W1, b1, W2, b2, alpha):
    out16 = _sc_call(edge_index, alpha)
    tr = out16[8:10]
    gates = out16[0:1].reshape(1, 1)
    return (tr, gates)
